# Initial kernel scaffold; baseline (speedup 1.0000x reference)
import functools

import jax
import jax.numpy as jnp
from jax import lax
from jax.experimental import pallas as pl
from jax.experimental.pallas import tpu as pltpu
from jax.experimental.pallas import tpu_sc as plsc

DIM = 32
NC = 2
NS = 16
NW = NC * NS
CHUNK = 128


def _make_gather(total_rows: int):
    rows_per_w = total_rows // NW
    chunks_per_w = rows_per_w // CHUNK
    mesh = plsc.VectorSubcoreMesh(core_axis_name="c", subcore_axis_name="s")

    @functools.partial(
        pl.kernel,
        out_type=jax.ShapeDtypeStruct((total_rows, DIM), jnp.float32),
        mesh=mesh,
        scratch_types=[
            pltpu.VMEM((chunks_per_w, CHUNK), jnp.int32),
            pltpu.VMEM((CHUNK, DIM), jnp.float32),
            pltpu.SemaphoreType.DMA,
        ],
    )
    def gather_kernel(idx_hbm, table_hbm, out_hbm, idx_v, buf_v, sem):
        wid = lax.axis_index("s") * NC + lax.axis_index("c")
        base = wid * rows_per_w
        pltpu.sync_copy(idx_hbm.at[wid], idx_v)

        def chunk_body(j, carry):
            pltpu.async_copy(table_hbm.at[idx_v.at[j]], buf_v, sem).wait()
            pltpu.sync_copy(buf_v, out_hbm.at[pl.ds(base + j * CHUNK, CHUNK)])
            return carry

        lax.fori_loop(0, chunks_per_w, chunk_body, 0)

    return gather_kernel


def kernel(x, E):
    b, t = x.shape
    total = b * t
    idx = x.reshape(NW, (total // NW) // CHUNK, CHUNK).astype(jnp.int32)
    out = _make_gather(total)(idx, E)
    return out.reshape(b, t, DIM)


# speedup vs baseline: 1.0213x; 1.0213x over previous
import functools

import jax
import jax.numpy as jnp
from jax import lax
from jax.experimental import pallas as pl
from jax.experimental.pallas import tpu as pltpu
from jax.experimental.pallas import tpu_sc as plsc

DIM = 32
NC = 2
NS = 16
NW = NC * NS
CHUNK = 128


def _make_gather(total_rows: int):
    rows_per_w = total_rows // NW
    chunks_per_w = rows_per_w // CHUNK
    mesh = plsc.VectorSubcoreMesh(core_axis_name="c", subcore_axis_name="s")

    @functools.partial(
        pl.kernel,
        out_type=jax.ShapeDtypeStruct((total_rows, DIM), jnp.float32),
        mesh=mesh,
        scratch_types=[
            pltpu.VMEM((chunks_per_w, CHUNK), jnp.int32),
            pltpu.VMEM((CHUNK, DIM), jnp.float32),
            pltpu.SemaphoreType.DMA,
        ],
        compiler_params=pltpu.CompilerParams(use_tc_tiling_on_sc=False),
    )
    def gather_kernel(idx_hbm, table_hbm, out_hbm, idx_v, buf_v, sem):
        wid = lax.axis_index("s") * NC + lax.axis_index("c")
        base = wid * rows_per_w
        pltpu.sync_copy(idx_hbm.at[wid], idx_v)

        def chunk_body(j, carry):
            pltpu.async_copy(table_hbm.at[idx_v.at[j]], buf_v, sem).wait()
            pltpu.sync_copy(buf_v, out_hbm.at[pl.ds(base + j * CHUNK, CHUNK)])
            return carry

        lax.fori_loop(0, chunks_per_w, chunk_body, 0)

    return gather_kernel


def kernel(x, E):
    b, t = x.shape
    total = b * t
    idx = x.reshape(NW, (total // NW) // CHUNK, CHUNK).astype(jnp.int32)
    out = _make_gather(total)(idx, E)
    return out.reshape(b, t, DIM)


# speedup vs baseline: 1.1122x; 1.0890x over previous
import functools

import jax
import jax.numpy as jnp
from jax import lax
from jax.experimental import pallas as pl
from jax.experimental.pallas import tpu as pltpu
from jax.experimental.pallas import tpu_sc as plsc

DIM = 32
NC = 2
NS = 16
NW = NC * NS
CHUNK = 128
K = 10
HALF = K * CHUNK


def _make_gather(total_rows: int):
    rows_per_w = total_rows // NW
    chunks_per_w = rows_per_w // CHUNK
    n_halves = chunks_per_w // K
    assert chunks_per_w % K == 0 and n_halves % 2 == 0 and n_halves >= 4
    mesh = plsc.VectorSubcoreMesh(core_axis_name="c", subcore_axis_name="s")

    @functools.partial(
        pl.kernel,
        out_type=jax.ShapeDtypeStruct((total_rows, DIM), jnp.float32),
        mesh=mesh,
        scratch_types=[
            pltpu.VMEM((chunks_per_w, CHUNK), jnp.int32),
            pltpu.VMEM((2 * HALF, DIM), jnp.float32),
            pltpu.SemaphoreType.DMA,
            pltpu.SemaphoreType.DMA,
            pltpu.SemaphoreType.DMA,
            pltpu.SemaphoreType.DMA,
        ],
        compiler_params=pltpu.CompilerParams(use_tc_tiling_on_sc=False),
    )
    def gather_kernel(idx_hbm, table_hbm, out_hbm, idx_v, bufs, gs0, gs1,
                      ws0, ws1):
        gsem = (gs0, gs1)
        wsem = (ws0, ws1)
        wid = lax.axis_index("s") * NC + lax.axis_index("c")
        base = wid * rows_per_w
        pltpu.sync_copy(idx_hbm.at[wid], idx_v)

        def out_slab(h):
            return out_hbm.at[pl.ds(base + h * HALF, HALF)]

        def half_buf(slot):
            return bufs.at[pl.ds(slot * HALF, HALF)]

        def fire_half(h, slot):
            for b in range(K):
                pltpu.async_copy(
                    table_hbm.at[idx_v.at[h * K + b]],
                    bufs.at[pl.ds(slot * HALF + b * CHUNK, CHUNK)],
                    gsem[slot],
                )

        def drain_gathers(h, slot):
            pltpu.make_async_copy(out_slab(h), half_buf(slot), gsem[slot]).wait()

        def process_half(h, slot):
            drain_gathers(h, slot)
            pltpu.async_copy(half_buf(slot), out_slab(h), wsem[slot])

        def refill_half(h, slot):
            pltpu.make_async_copy(half_buf(slot), out_slab(h), wsem[slot]).wait()
            fire_half(h, slot)

        fire_half(0, 0)
        fire_half(1, 1)

        def body(i, carry):
            h0 = 2 * i
            process_half(h0, 0)
            refill_half(h0 + 2, 0)
            process_half(h0 + 1, 1)
            refill_half(h0 + 3, 1)
            return carry

        lax.fori_loop(0, n_halves // 2 - 1, body, 0)

        h_last = n_halves - 2
        process_half(h_last, 0)
        process_half(h_last + 1, 1)
        pltpu.make_async_copy(half_buf(0), out_slab(h_last), wsem[0]).wait()
        pltpu.make_async_copy(half_buf(1), out_slab(h_last + 1), wsem[1]).wait()

    return gather_kernel


def kernel(x, E):
    b, t = x.shape
    total = b * t
    idx = x.reshape(NW, (total // NW) // CHUNK, CHUNK).astype(jnp.int32)
    out = _make_gather(total)(idx, E)
    return out.reshape(b, t, DIM)


# speedup vs baseline: 1.8032x; 1.6212x over previous
import functools

import jax
import jax.numpy as jnp
from jax import lax
from jax.experimental import pallas as pl
from jax.experimental.pallas import tpu as pltpu
from jax.experimental.pallas import tpu_sc as plsc

DIM = 32
NC = 2
NS = 16
NW = NC * NS
K = 16


def _make_gather(B: int, T: int):
    rows_per_w = B // NW
    n_halves = rows_per_w // K
    assert rows_per_w % K == 0 and n_halves % 2 == 0 and n_halves >= 4
    mesh = plsc.VectorSubcoreMesh(core_axis_name="c", subcore_axis_name="s")

    @functools.partial(
        pl.kernel,
        out_type=jax.ShapeDtypeStruct((B, T, DIM), jnp.float32),
        mesh=mesh,
        scratch_types=[
            pltpu.VMEM((rows_per_w, T), jnp.int32),
            pltpu.VMEM((2 * K, T, DIM), jnp.float32),
            pltpu.SemaphoreType.DMA,
            pltpu.SemaphoreType.DMA,
            pltpu.SemaphoreType.DMA,
            pltpu.SemaphoreType.DMA,
        ],
        compiler_params=pltpu.CompilerParams(use_tc_tiling_on_sc=False),
    )
    def gather_kernel(idx_hbm, table_hbm, out_hbm, idx_v, bufs, gs0, gs1,
                      ws0, ws1):
        gsem = (gs0, gs1)
        wsem = (ws0, ws1)
        wid = lax.axis_index("s") * NC + lax.axis_index("c")
        base = wid * rows_per_w
        pltpu.sync_copy(idx_hbm.at[wid], idx_v)

        def out_slab(h):
            return out_hbm.at[pl.ds(base + h * K, K)]

        def half_buf(slot):
            return bufs.at[pl.ds(slot * K, K)]

        def fire_half(h, slot):
            for c in range(K):
                pltpu.async_copy(
                    table_hbm.at[idx_v.at[h * K + c]],
                    bufs.at[slot * K + c],
                    gsem[slot],
                )

        def process_half(h, slot):
            for c in range(K):
                pltpu.make_async_copy(
                    table_hbm.at[idx_v.at[h * K + c]],
                    bufs.at[slot * K + c],
                    gsem[slot],
                ).wait()
            pltpu.async_copy(half_buf(slot), out_slab(h), wsem[slot])

        def refill_half(h, slot):
            pltpu.make_async_copy(half_buf(slot), out_slab(h), wsem[slot]).wait()
            fire_half(h, slot)

        fire_half(0, 0)
        fire_half(1, 1)

        def body(i, carry):
            h0 = 2 * i
            process_half(h0, 0)
            refill_half(h0 + 2, 0)
            process_half(h0 + 1, 1)
            refill_half(h0 + 3, 1)
            return carry

        lax.fori_loop(0, n_halves // 2 - 1, body, 0)

        h_last = n_halves - 2
        process_half(h_last, 0)
        process_half(h_last + 1, 1)
        pltpu.make_async_copy(half_buf(0), out_slab(h_last), wsem[0]).wait()
        pltpu.make_async_copy(half_buf(1), out_slab(h_last + 1), wsem[1]).wait()

    return gather_kernel


def kernel(x, E):
    b, t = x.shape
    idx = x.reshape(NW, b // NW, t).astype(jnp.int32)
    return _make_gather(b, t)(idx, E)
